# Initial kernel scaffold; baseline (speedup 1.0000x reference)
import jax
import jax.numpy as jnp
from jax.experimental import pallas as pl


def kernel(x, nbr1, nbr2, nbr3, W1_0, b1_0, W1_1, b1_1, g1, be1, W2_0, b2_0, W2_1, b2_1, g2, be2, W3_0, b3_0, W3_1, b3_1, g3, be3, ffw1, ffb1, ffw2, ffb2, ffw3, ffb3):
    raise NotImplementedError("write your pallas kernel here")


# speedup vs baseline: 10.8136x; 10.8136x over previous
import functools

import jax
import jax.numpy as jnp
from jax import lax
from jax.experimental import pallas as pl
from jax.experimental.pallas import tpu as pltpu
from jax.experimental.pallas import tpu_sc as plsc

_E0, _E1, _E2, _E3 = 160000, 144000, 129600, 116640
_BE = 800
_NW = 32
_MAX_CHUNK_WORDS = 46080


def _pick_chunk(bpw, d):
    best = 0
    for ch in range(8, bpw + 1, 8):
        if bpw % ch == 0 and ch * d <= _MAX_CHUNK_WORDS:
            best = ch
    return best


def _plan_pad(b, d):
    best = (0, 0)
    bp = ((b + 255) // 256) * 256
    for _ in range(64):
        ch = _pick_chunk(bp // _NW, d)
        if ch >= 256:
            return bp, ch
        if ch > best[1]:
            best = (bp, ch)
        bp += 256
    return best


@functools.lru_cache(maxsize=None)
def _sc_gather_call(v, d, b):
    assert b % (8 * _NW) == 0
    bpw = b // _NW
    ch = _pick_chunk(bpw, d)
    assert ch > 0, (v, d, b)
    nchunks = bpw // ch
    mesh = plsc.VectorSubcoreMesh(core_axis_name="c", subcore_axis_name="s")

    @functools.partial(
        pl.kernel,
        mesh=mesh,
        out_type=jax.ShapeDtypeStruct((b, d), jnp.float32),
        scratch_types=[
            pltpu.VMEM((ch,), jnp.int32),
            pltpu.VMEM((ch, d), jnp.float32),
            pltpu.SemaphoreType.DMA,
        ],
        compiler_params=pltpu.CompilerParams(use_tc_tiling_on_sc=False),
    )
    def k(table_hbm, idx_hbm, out_hbm, idx_v, rows_v, sem):
        wid = lax.axis_index("s") * 2 + lax.axis_index("c")
        base = wid * bpw

        def body(i, carry):
            off = base + i * ch
            pltpu.sync_copy(idx_hbm.at[pl.ds(off, ch)], idx_v)
            pltpu.async_copy(table_hbm.at[idx_v], rows_v, sem).wait()
            pltpu.sync_copy(rows_v, out_hbm.at[pl.ds(off, ch)])
            return carry

        lax.fori_loop(0, nchunks, body, 0)

    return k


def _gather_rows(table, idx):
    v, d = table.shape
    b = idx.shape[0]
    if b % 256 == 0 and _pick_chunk(b // _NW, d) > 0:
        bp = b
    else:
        bp, _ = _plan_pad(b, d)
    if bp != b:
        pad = jnp.arange(bp - b, dtype=jnp.int32) % v
        idx = jnp.concatenate([idx, pad])
    out = _sc_gather_call(v, d, bp)(table, idx)
    return out[:b] if bp != b else out


def _conv0_body(nsteps, x_ref, a_ref, b_ref, c_ref, d_ref, wf_ref, bias_ref,
                out_ref, mom_ref):
    i = pl.program_id(0)
    x = x_ref[...]
    a, bb, cc, dd = a_ref[...], b_ref[...], c_ref[...], d_ref[...]
    feats = jnp.concatenate(
        [x, jnp.abs(a - cc), a + cc, jnp.abs(bb - dd), bb + dd], axis=1)
    out = jnp.dot(feats, wf_ref[...], preferred_element_type=jnp.float32,
                  precision=lax.Precision.HIGHEST) + bias_ref[...]
    out_ref[...] = out
    r = jnp.maximum(out, 0.0)

    @pl.when(i == 0)
    def _():
        mom_ref[...] = jnp.zeros_like(mom_ref)

    mom_ref[0:1, :] += jnp.sum(r, axis=0, keepdims=True)
    mom_ref[1:2, :] += jnp.sum(r * r, axis=0, keepdims=True)


def _conv1_body(e_edges, nsteps, x0_ref, a_ref, b_ref, c_ref, d_ref, mom_ref,
                g_ref, be_ref, wf_ref, bias_ref, out_ref, nrm_ref, cs_ref):
    i = pl.program_id(0)
    inv_e = 1.0 / e_edges
    m = mom_ref[0:1, :] * inv_e
    q = mom_ref[1:2, :] * inv_e
    var = q - m * m
    s = g_ref[...] * lax.rsqrt(var + 1e-5)
    t = be_ref[...] - m * s

    x0 = x0_ref[...]
    h = jnp.maximum(x0, 0.0) * s + t
    ha = jnp.maximum(a_ref[...], 0.0) * s + t
    hb = jnp.maximum(b_ref[...], 0.0) * s + t
    hc = jnp.maximum(c_ref[...], 0.0) * s + t
    hd = jnp.maximum(d_ref[...], 0.0) * s + t
    feats = jnp.concatenate(
        [h, jnp.abs(ha - hc), ha + hc, jnp.abs(hb - hd), hb + hd], axis=1)
    out = jnp.dot(feats, wf_ref[...], preferred_element_type=jnp.float32,
                  precision=lax.Precision.HIGHEST) + bias_ref[...] + x0
    out = jnp.maximum(out, 0.0)
    out_ref[...] = out
    nrm_ref[...] = jnp.sqrt(jnp.sum(out * out, axis=1))[:, None]

    @pl.when(i == 0)
    def _():
        cs_ref[...] = jnp.zeros_like(cs_ref)

    cs_ref[0:1, :] += jnp.sum(out, axis=0, keepdims=True)


def _conv0(xe, g4, wf, bias, e):
    c = xe.shape[1]
    o = wf.shape[1]
    n = e // _BE
    blk = lambda j: pl.BlockSpec((_BE, c), lambda i, j=j: (j * n + i, 0))
    return pl.pallas_call(
        functools.partial(_conv0_body, n),
        grid=(n,),
        in_specs=[
            pl.BlockSpec((_BE, c), lambda i: (i, 0)),
            blk(0), blk(1), blk(2), blk(3),
            pl.BlockSpec((5 * c, o), lambda i: (0, 0)),
            pl.BlockSpec((1, o), lambda i: (0, 0)),
        ],
        out_specs=[
            pl.BlockSpec((_BE, o), lambda i: (i, 0)),
            pl.BlockSpec((8, o), lambda i: (0, 0)),
        ],
        out_shape=[
            jax.ShapeDtypeStruct((e, o), jnp.float32),
            jax.ShapeDtypeStruct((8, o), jnp.float32),
        ],
        compiler_params=pltpu.CompilerParams(
            dimension_semantics=("arbitrary",)),
    )(xe, g4, g4, g4, g4, wf, bias)


def _conv1(x0, g4, mom, gam, bet, wf, bias, e):
    o = x0.shape[1]
    o2 = wf.shape[1]
    n = e // _BE
    blk = lambda j: pl.BlockSpec((_BE, o), lambda i, j=j: (j * n + i, 0))
    return pl.pallas_call(
        functools.partial(_conv1_body, float(e), n),
        grid=(n,),
        in_specs=[
            pl.BlockSpec((_BE, o), lambda i: (i, 0)),
            blk(0), blk(1), blk(2), blk(3),
            pl.BlockSpec((8, o), lambda i: (0, 0)),
            pl.BlockSpec((1, o), lambda i: (0, 0)),
            pl.BlockSpec((1, o), lambda i: (0, 0)),
            pl.BlockSpec((5 * o, o2), lambda i: (0, 0)),
            pl.BlockSpec((1, o2), lambda i: (0, 0)),
        ],
        out_specs=[
            pl.BlockSpec((_BE, o2), lambda i: (i, 0)),
            pl.BlockSpec((_BE, 1), lambda i: (i, 0)),
            pl.BlockSpec((8, o2), lambda i: (0, 0)),
        ],
        out_shape=[
            jax.ShapeDtypeStruct((e, o2), jnp.float32),
            jax.ShapeDtypeStruct((e, 1), jnp.float32),
            jax.ShapeDtypeStruct((8, o2), jnp.float32),
        ],
        compiler_params=pltpu.CompilerParams(
            dimension_semantics=("arbitrary",)),
    )(x0, g4, g4, g4, g4, mom, gam, bet, wf, bias)


def _head_body(drop_ref, cs_ref, w1_ref, b1_ref, w2_ref, b2_ref, w3_ref,
               b3_ref, out_ref):
    dsum = jnp.sum(drop_ref[...], axis=0, keepdims=True)
    mean = (cs_ref[0:1, :] - dsum) * (1.0 / _E3)
    hp = lax.Precision.HIGHEST
    z = jnp.maximum(jnp.dot(mean, w1_ref[...], preferred_element_type=jnp.float32,
                            precision=hp) + b1_ref[...], 0.0)
    z = jnp.maximum(jnp.dot(z, w2_ref[...], preferred_element_type=jnp.float32,
                            precision=hp) + b2_ref[...], 0.0)
    z = jnp.dot(z, w3_ref[...], preferred_element_type=jnp.float32,
                precision=hp) + b3_ref[...]
    out_ref[...] = z


def _head(drop_rows, cs, w1t, b1, w2t, b2, w3t, b3):
    nd = drop_rows.shape[0]
    return pl.pallas_call(
        _head_body,
        grid=(1,),
        in_specs=[
            pl.BlockSpec((nd, 128), lambda i: (0, 0)),
            pl.BlockSpec((8, 128), lambda i: (0, 0)),
            pl.BlockSpec((128, 128), lambda i: (0, 0)),
            pl.BlockSpec((1, 128), lambda i: (0, 0)),
            pl.BlockSpec((128, 64), lambda i: (0, 0)),
            pl.BlockSpec((1, 64), lambda i: (0, 0)),
            pl.BlockSpec((64, 32), lambda i: (0, 0)),
            pl.BlockSpec((1, 32), lambda i: (0, 0)),
        ],
        out_specs=pl.BlockSpec((1, 32), lambda i: (0, 0)),
        out_shape=jax.ShapeDtypeStruct((1, 32), jnp.float32),
    )(drop_rows, cs, w1t, b1, w2t, b2, w3t, b3)


def _fold_w(w):
    o, c, _ = w.shape
    return jnp.transpose(w, (2, 1, 0)).reshape(5 * c, o)


def _res_block(xe, nbr, wf0, b0, wf1, b1, gam, bet, e):
    idx = jnp.transpose(nbr[0]).reshape(-1).astype(jnp.int32)
    c = xe.shape[1]
    g0 = _gather_rows(xe, idx)
    out0, mom = _conv0(xe, g0, wf0, b0, e)
    g1 = _gather_rows(out0, idx)
    return _conv1(out0, g1, mom, gam, bet, wf1, b1, e)


def kernel(x, nbr1, nbr2, nbr3, W1_0, b1_0, W1_1, b1_1, g1, be1,
           W2_0, b2_0, W2_1, b2_1, g2, be2, W3_0, b3_0, W3_1, b3_1, g3, be3,
           ffw1, ffb1, ffw2, ffb2, ffw3, ffb3):
    f32 = jnp.float32
    x0 = jnp.pad(x[0].T.astype(f32), ((0, 0), (0, 13)))
    w10 = jnp.pad(W1_0, ((0, 0), (0, 13), (0, 0)))
    wf10 = _fold_w(w10)
    row = lambda v: v.reshape(1, -1).astype(f32)

    out1, nrm1, _ = _res_block(x0, nbr1, wf10, row(b1_0), _fold_w(W1_1),
                               row(b1_1), row(g1), row(be1), _E0)
    _, i1 = lax.top_k(nrm1[:, 0], _E1)
    x2 = _gather_rows(out1, i1.astype(jnp.int32))

    out2, nrm2, _ = _res_block(x2, nbr2, _fold_w(W2_0), row(b2_0),
                               _fold_w(W2_1), row(b2_1), row(g2), row(be2), _E1)
    _, i2 = lax.top_k(nrm2[:, 0], _E2)
    x3 = _gather_rows(out2, i2.astype(jnp.int32))

    out3, nrm3, cs3 = _res_block(x3, nbr3, _fold_w(W3_0), row(b3_0),
                                 _fold_w(W3_1), row(b3_1), row(g3), row(be3),
                                 _E2)
    _, i3 = lax.top_k(nrm3[:, 0], _E2)
    dropped = i3[_E3:].astype(jnp.int32)
    drop_rows = _gather_rows(out3, dropped)

    return _head(drop_rows, cs3, ffw1.T.astype(f32), row(ffb1),
                 ffw2.T.astype(f32), row(ffb2), ffw3.T.astype(f32), row(ffb3))


# speedup vs baseline: 12.9207x; 1.1949x over previous
import functools

import jax
import jax.numpy as jnp
from jax import lax
from jax.experimental import pallas as pl
from jax.experimental.pallas import tpu as pltpu
from jax.experimental.pallas import tpu_sc as plsc

_E0, _E1, _E2, _E3 = 160000, 144000, 129600, 116640
_BE = 1600
_NW = 32
_MAX_CHUNK_WORDS = 46080


def _pick_chunk(bpw, d):
    best = 0
    for ch in range(8, bpw + 1, 8):
        if bpw % ch == 0 and ch * d <= _MAX_CHUNK_WORDS:
            best = ch
    return best


def _plan_pad(b, d):
    best = (0, 0)
    bp = ((b + 255) // 256) * 256
    for _ in range(64):
        ch = _pick_chunk(bp // _NW, d)
        if ch >= 256:
            return bp, ch
        if ch > best[1]:
            best = (bp, ch)
        bp += 256
    return best


def _dot3(a, b):
    ah = a.astype(jnp.bfloat16)
    al = (a - ah.astype(jnp.float32)).astype(jnp.bfloat16)
    bh = b.astype(jnp.bfloat16)
    bl = (b - bh.astype(jnp.float32)).astype(jnp.bfloat16)
    kw = dict(preferred_element_type=jnp.float32)
    return (jnp.dot(ah, bl, **kw) + jnp.dot(al, bh, **kw)) + jnp.dot(ah, bh, **kw)


@functools.lru_cache(maxsize=None)
def _sc_gather_call(v, d, b, tc_tiling):
    assert b % (8 * _NW) == 0
    bpw = b // _NW
    ch = _pick_chunk(bpw, d)
    assert ch > 0, (v, d, b)
    nchunks = bpw // ch
    mesh = plsc.VectorSubcoreMesh(core_axis_name="c", subcore_axis_name="s")

    @functools.partial(
        pl.kernel,
        mesh=mesh,
        out_type=jax.ShapeDtypeStruct((b, d), jnp.float32),
        scratch_types=[
            pltpu.VMEM((ch,), jnp.int32),
            pltpu.VMEM((ch, d), jnp.float32),
            pltpu.SemaphoreType.DMA,
        ],
        compiler_params=pltpu.CompilerParams(use_tc_tiling_on_sc=tc_tiling),
    )
    def k(table_hbm, idx_hbm, out_hbm, idx_v, rows_v, sem):
        wid = lax.axis_index("s") * 2 + lax.axis_index("c")
        base = wid * bpw

        def body(i, carry):
            off = base + i * ch
            pltpu.sync_copy(idx_hbm.at[pl.ds(off, ch)], idx_v)
            pltpu.async_copy(table_hbm.at[idx_v], rows_v, sem).wait()
            pltpu.sync_copy(rows_v, out_hbm.at[pl.ds(off, ch)])
            return carry

        lax.fori_loop(0, nchunks, body, 0)

    return k


def _gather_rows(table, idx):
    v, d = table.shape
    b = idx.shape[0]
    if b % 256 == 0 and _pick_chunk(b // _NW, d) > 0:
        bp = b
    else:
        bp, _ = _plan_pad(b, d)
    if bp != b:
        pad = jnp.arange(bp - b, dtype=jnp.int32) % v
        idx = jnp.concatenate([idx, pad])
    return _sc_gather_call(v, d, bp, d % 128 == 0)(table, idx)


def _conv0_body(nsteps, x_ref, a_ref, b_ref, c_ref, d_ref, wf_ref, bias_ref,
                out_ref, mom_ref):
    i = pl.program_id(0)
    x = x_ref[...]
    a, bb, cc, dd = a_ref[...], b_ref[...], c_ref[...], d_ref[...]
    feats = jnp.concatenate(
        [x, jnp.abs(a - cc), a + cc, jnp.abs(bb - dd), bb + dd], axis=1)
    out = _dot3(feats, wf_ref[...]) + bias_ref[...]
    out_ref[...] = out
    r = jnp.maximum(out, 0.0)

    @pl.when(i == 0)
    def _():
        mom_ref[...] = jnp.zeros_like(mom_ref)

    mom_ref[0:1, :] += jnp.sum(r, axis=0, keepdims=True)
    mom_ref[1:2, :] += jnp.sum(r * r, axis=0, keepdims=True)


def _conv1_body(e_edges, nsteps, x0_ref, a_ref, b_ref, c_ref, d_ref, mom_ref,
                g_ref, be_ref, wf_ref, bias_ref, out_ref, nrm_ref, cs_ref):
    i = pl.program_id(0)
    inv_e = 1.0 / e_edges
    m = mom_ref[0:1, :] * inv_e
    q = mom_ref[1:2, :] * inv_e
    var = q - m * m
    s = g_ref[...] * lax.rsqrt(var + 1e-5)
    t = be_ref[...] - m * s

    x0 = x0_ref[...]
    h = jnp.maximum(x0, 0.0) * s + t
    ha = jnp.maximum(a_ref[...], 0.0) * s + t
    hb = jnp.maximum(b_ref[...], 0.0) * s + t
    hc = jnp.maximum(c_ref[...], 0.0) * s + t
    hd = jnp.maximum(d_ref[...], 0.0) * s + t
    feats = jnp.concatenate(
        [h, jnp.abs(ha - hc), ha + hc, jnp.abs(hb - hd), hb + hd], axis=1)
    out = _dot3(feats, wf_ref[...]) + bias_ref[...] + x0
    out = jnp.maximum(out, 0.0)
    out_ref[...] = out
    nrm_ref[...] = jnp.sqrt(jnp.sum(out * out, axis=1))[:, None]

    @pl.when(i == 0)
    def _():
        cs_ref[...] = jnp.zeros_like(cs_ref)

    cs_ref[0:1, :] += jnp.sum(out, axis=0, keepdims=True)


def _conv0(xe, g4, wf, bias, e):
    c = xe.shape[1]
    o = wf.shape[1]
    n = e // _BE
    blk = lambda j: pl.BlockSpec((_BE, c), lambda i, j=j: (j * n + i, 0))
    return pl.pallas_call(
        functools.partial(_conv0_body, n),
        grid=(n,),
        in_specs=[
            pl.BlockSpec((_BE, c), lambda i: (i, 0)),
            blk(0), blk(1), blk(2), blk(3),
            pl.BlockSpec((5 * c, o), lambda i: (0, 0)),
            pl.BlockSpec((1, o), lambda i: (0, 0)),
        ],
        out_specs=[
            pl.BlockSpec((_BE, o), lambda i: (i, 0)),
            pl.BlockSpec((8, o), lambda i: (0, 0)),
        ],
        out_shape=[
            jax.ShapeDtypeStruct((e, o), jnp.float32),
            jax.ShapeDtypeStruct((8, o), jnp.float32),
        ],
        compiler_params=pltpu.CompilerParams(
            dimension_semantics=("arbitrary",)),
    )(xe, g4, g4, g4, g4, wf, bias)


def _conv1(x0, g4, mom, gam, bet, wf, bias, e):
    o = x0.shape[1]
    o2 = wf.shape[1]
    n = e // _BE
    blk = lambda j: pl.BlockSpec((_BE, o), lambda i, j=j: (j * n + i, 0))
    return pl.pallas_call(
        functools.partial(_conv1_body, float(e), n),
        grid=(n,),
        in_specs=[
            pl.BlockSpec((_BE, o), lambda i: (i, 0)),
            blk(0), blk(1), blk(2), blk(3),
            pl.BlockSpec((8, o), lambda i: (0, 0)),
            pl.BlockSpec((1, o), lambda i: (0, 0)),
            pl.BlockSpec((1, o), lambda i: (0, 0)),
            pl.BlockSpec((5 * o, o2), lambda i: (0, 0)),
            pl.BlockSpec((1, o2), lambda i: (0, 0)),
        ],
        out_specs=[
            pl.BlockSpec((_BE, o2), lambda i: (i, 0)),
            pl.BlockSpec((_BE, 1), lambda i: (i, 0)),
            pl.BlockSpec((8, o2), lambda i: (0, 0)),
        ],
        out_shape=[
            jax.ShapeDtypeStruct((e, o2), jnp.float32),
            jax.ShapeDtypeStruct((e, 1), jnp.float32),
            jax.ShapeDtypeStruct((8, o2), jnp.float32),
        ],
        compiler_params=pltpu.CompilerParams(
            dimension_semantics=("arbitrary",)),
    )(x0, g4, g4, g4, g4, mom, gam, bet, wf, bias)


def _head_body(nd_real, drop_ref, cs_ref, w1_ref, b1_ref, w2_ref, b2_ref,
               w3_ref, b3_ref, out_ref):
    nd_pad = drop_ref.shape[0]
    rows = lax.broadcasted_iota(jnp.int32, (nd_pad, 1), 0)
    mask = (rows < nd_real).astype(jnp.float32)
    dsum = jnp.sum(drop_ref[...] * mask, axis=0, keepdims=True)
    mean = (cs_ref[0:1, :] - dsum) * (1.0 / _E3)
    z = jnp.maximum(_dot3(mean, w1_ref[...]) + b1_ref[...], 0.0)
    z = jnp.maximum(_dot3(z, w2_ref[...]) + b2_ref[...], 0.0)
    z = _dot3(z, w3_ref[...]) + b3_ref[...]
    out_ref[...] = z


def _head(drop_rows, nd_real, cs, w1t, b1, w2t, b2, w3t, b3):
    nd = drop_rows.shape[0]
    return pl.pallas_call(
        functools.partial(_head_body, nd_real),
        grid=(1,),
        in_specs=[
            pl.BlockSpec((nd, 128), lambda i: (0, 0)),
            pl.BlockSpec((8, 128), lambda i: (0, 0)),
            pl.BlockSpec((128, 128), lambda i: (0, 0)),
            pl.BlockSpec((1, 128), lambda i: (0, 0)),
            pl.BlockSpec((128, 64), lambda i: (0, 0)),
            pl.BlockSpec((1, 64), lambda i: (0, 0)),
            pl.BlockSpec((64, 32), lambda i: (0, 0)),
            pl.BlockSpec((1, 32), lambda i: (0, 0)),
        ],
        out_specs=pl.BlockSpec((1, 32), lambda i: (0, 0)),
        out_shape=jax.ShapeDtypeStruct((1, 32), jnp.float32),
    )(drop_rows, cs, w1t, b1, w2t, b2, w3t, b3)


def _fold_w(w):
    o, c, _ = w.shape
    return jnp.transpose(w, (2, 1, 0)).reshape(5 * c, o)


def _res_block(xe, nbr, wf0, b0, wf1, b1, gam, bet, e):
    idx = jnp.transpose(nbr[0]).reshape(-1).astype(jnp.int32)
    c = xe.shape[1]
    g0 = _gather_rows(xe, idx)
    out0, mom = _conv0(xe, g0, wf0, b0, e)
    g1 = _gather_rows(out0, idx)
    return _conv1(out0, g1, mom, gam, bet, wf1, b1, e)


def kernel(x, nbr1, nbr2, nbr3, W1_0, b1_0, W1_1, b1_1, g1, be1,
           W2_0, b2_0, W2_1, b2_1, g2, be2, W3_0, b3_0, W3_1, b3_1, g3, be3,
           ffw1, ffb1, ffw2, ffb2, ffw3, ffb3):
    f32 = jnp.float32
    x0 = jnp.pad(x[0].T.astype(f32), ((0, 0), (0, 13)))
    w10 = jnp.pad(W1_0, ((0, 0), (0, 13), (0, 0)))
    wf10 = _fold_w(w10)
    row = lambda v: v.reshape(1, -1).astype(f32)

    out1, nrm1, _ = _res_block(x0, nbr1, wf10, row(b1_0), _fold_w(W1_1),
                               row(b1_1), row(g1), row(be1), _E0)
    _, i1 = lax.top_k(nrm1[:, 0], _E1)
    x2 = _gather_rows(out1, i1.astype(jnp.int32))

    out2, nrm2, _ = _res_block(x2, nbr2, _fold_w(W2_0), row(b2_0),
                               _fold_w(W2_1), row(b2_1), row(g2), row(be2), _E1)
    _, i2 = lax.top_k(nrm2[:, 0], _E2)
    x3 = _gather_rows(out2, i2.astype(jnp.int32))

    out3, nrm3, cs3 = _res_block(x3, nbr3, _fold_w(W3_0), row(b3_0),
                                 _fold_w(W3_1), row(b3_1), row(g3), row(be3),
                                 _E2)
    _, i3 = lax.top_k(nrm3[:, 0], _E2)
    dropped = i3[_E3:].astype(jnp.int32)
    drop_rows = _gather_rows(out3, dropped)

    return _head(drop_rows, _E2 - _E3, cs3, ffw1.T.astype(f32), row(ffb1),
                 ffw2.T.astype(f32), row(ffb2), ffw3.T.astype(f32), row(ffb3))


# speedup vs baseline: 13.1590x; 1.0184x over previous
import functools

import jax
import jax.numpy as jnp
from jax import lax
from jax.experimental import pallas as pl
from jax.experimental.pallas import tpu as pltpu
from jax.experimental.pallas import tpu_sc as plsc

_E0, _E1, _E2, _E3 = 160000, 144000, 129600, 116640
_BE = 1600
_NW = 32
_MAX_CHUNK_WORDS = 57344


def _pick_chunk(bpw, d):
    best = 0
    for ch in range(8, bpw + 1, 8):
        if bpw % ch == 0 and ch * d <= _MAX_CHUNK_WORDS:
            best = ch
    return best


def _plan_pad(b, d):
    best = (0, 0)
    bp = ((b + 255) // 256) * 256
    for _ in range(64):
        ch = _pick_chunk(bp // _NW, d)
        if ch >= 256:
            return bp, ch
        if ch > best[1]:
            best = (bp, ch)
        bp += 256
    return best


def _dot3(a, b):
    ah = a.astype(jnp.bfloat16)
    al = (a - ah.astype(jnp.float32)).astype(jnp.bfloat16)
    bh = b.astype(jnp.bfloat16)
    bl = (b - bh.astype(jnp.float32)).astype(jnp.bfloat16)
    kw = dict(preferred_element_type=jnp.float32)
    return (jnp.dot(ah, bl, **kw) + jnp.dot(al, bh, **kw)) + jnp.dot(ah, bh, **kw)


@functools.lru_cache(maxsize=None)
def _sc_gather_call(v, d, b, tc_tiling):
    assert b % (8 * _NW) == 0
    bpw = b // _NW
    ch = _pick_chunk(bpw, d)
    assert ch > 0, (v, d, b)
    nchunks = bpw // ch
    mesh = plsc.VectorSubcoreMesh(core_axis_name="c", subcore_axis_name="s")

    @functools.partial(
        pl.kernel,
        mesh=mesh,
        out_type=jax.ShapeDtypeStruct((b, d), jnp.float32),
        scratch_types=[
            pltpu.VMEM((2 * ch,), jnp.int32),
            pltpu.VMEM((2, ch, d), jnp.float32),
            pltpu.SemaphoreType.DMA,
            pltpu.SemaphoreType.DMA,
        ],
        compiler_params=pltpu.CompilerParams(use_tc_tiling_on_sc=tc_tiling),
    )
    def k(table_hbm, idx_hbm, out_hbm, idx_v, rows_v, gsem, wsem):
        wid = lax.axis_index("s") * 2 + lax.axis_index("c")
        base = wid * bpw
        pltpu.sync_copy(idx_hbm.at[pl.ds(base, ch)], idx_v.at[pl.ds(0, ch)])
        pltpu.async_copy(table_hbm.at[idx_v.at[pl.ds(0, ch)]], rows_v.at[0], gsem)

        def body(i, carry):
            cur = lax.rem(i, 2)
            nxt = lax.rem(i + 1, 2)
            off = base + i * ch
            pltpu.make_async_copy(
                table_hbm.at[idx_v.at[pl.ds(cur * ch, ch)]], rows_v.at[cur],
                gsem).wait()

            @pl.when(i >= 1)
            def _():
                pltpu.make_async_copy(
                    rows_v.at[nxt], out_hbm.at[pl.ds(off - ch, ch)], wsem).wait()

            @pl.when(i + 1 < nchunks)
            def _():
                pltpu.sync_copy(idx_hbm.at[pl.ds(off + ch, ch)],
                                idx_v.at[pl.ds(nxt * ch, ch)])
                pltpu.async_copy(table_hbm.at[idx_v.at[pl.ds(nxt * ch, ch)]],
                                 rows_v.at[nxt], gsem)

            pltpu.async_copy(rows_v.at[cur], out_hbm.at[pl.ds(off, ch)], wsem)
            return carry

        lax.fori_loop(0, nchunks, body, 0)
        last = nchunks - 1
        pltpu.make_async_copy(
            rows_v.at[last % 2],
            out_hbm.at[pl.ds(base + last * ch, ch)], wsem).wait()

    return k


def _gather_rows(table, idx):
    v, d = table.shape
    b = idx.shape[0]
    if b % 256 == 0 and _pick_chunk(b // _NW, d) > 0:
        bp = b
    else:
        bp, _ = _plan_pad(b, d)
    if bp != b:
        pad = jnp.arange(bp - b, dtype=jnp.int32) % v
        idx = jnp.concatenate([idx, pad])
    return _sc_gather_call(v, d, bp, d % 128 == 0)(table, idx)


def _conv0_body(nsteps, x_ref, a_ref, b_ref, c_ref, d_ref, wf_ref, bias_ref,
                out_ref, mom_ref):
    i = pl.program_id(0)
    x = x_ref[...]
    a, bb, cc, dd = a_ref[...], b_ref[...], c_ref[...], d_ref[...]
    feats = jnp.concatenate(
        [x, jnp.abs(a - cc), a + cc, jnp.abs(bb - dd), bb + dd], axis=1)
    out = _dot3(feats, wf_ref[...]) + bias_ref[...]
    out_ref[...] = out
    r = jnp.maximum(out, 0.0)

    @pl.when(i == 0)
    def _():
        mom_ref[...] = jnp.zeros_like(mom_ref)

    mom_ref[0:1, :] += jnp.sum(r, axis=0, keepdims=True)
    mom_ref[1:2, :] += jnp.sum(r * r, axis=0, keepdims=True)


def _conv1_body(e_edges, nsteps, x0_ref, a_ref, b_ref, c_ref, d_ref, mom_ref,
                g_ref, be_ref, wf_ref, bias_ref, out_ref, nrm_ref, cs_ref):
    i = pl.program_id(0)
    inv_e = 1.0 / e_edges
    m = mom_ref[0:1, :] * inv_e
    q = mom_ref[1:2, :] * inv_e
    var = q - m * m
    s = g_ref[...] * lax.rsqrt(var + 1e-5)
    t = be_ref[...] - m * s

    x0 = x0_ref[...]
    h = jnp.maximum(x0, 0.0) * s + t
    ha = jnp.maximum(a_ref[...], 0.0) * s + t
    hb = jnp.maximum(b_ref[...], 0.0) * s + t
    hc = jnp.maximum(c_ref[...], 0.0) * s + t
    hd = jnp.maximum(d_ref[...], 0.0) * s + t
    feats = jnp.concatenate(
        [h, jnp.abs(ha - hc), ha + hc, jnp.abs(hb - hd), hb + hd], axis=1)
    out = _dot3(feats, wf_ref[...]) + bias_ref[...] + x0
    out = jnp.maximum(out, 0.0)
    out_ref[...] = out
    nrm_ref[...] = jnp.sqrt(jnp.sum(out * out, axis=1))[:, None]

    @pl.when(i == 0)
    def _():
        cs_ref[...] = jnp.zeros_like(cs_ref)

    cs_ref[0:1, :] += jnp.sum(out, axis=0, keepdims=True)


def _conv0(xe, g4, wf, bias, e):
    c = xe.shape[1]
    o = wf.shape[1]
    n = e // _BE
    blk = lambda j: pl.BlockSpec((_BE, c), lambda i, j=j: (j * n + i, 0))
    return pl.pallas_call(
        functools.partial(_conv0_body, n),
        grid=(n,),
        in_specs=[
            pl.BlockSpec((_BE, c), lambda i: (i, 0)),
            blk(0), blk(1), blk(2), blk(3),
            pl.BlockSpec((5 * c, o), lambda i: (0, 0)),
            pl.BlockSpec((1, o), lambda i: (0, 0)),
        ],
        out_specs=[
            pl.BlockSpec((_BE, o), lambda i: (i, 0)),
            pl.BlockSpec((8, o), lambda i: (0, 0)),
        ],
        out_shape=[
            jax.ShapeDtypeStruct((e, o), jnp.float32),
            jax.ShapeDtypeStruct((8, o), jnp.float32),
        ],
        compiler_params=pltpu.CompilerParams(
            dimension_semantics=("arbitrary",)),
    )(xe, g4, g4, g4, g4, wf, bias)


def _conv1(x0, g4, mom, gam, bet, wf, bias, e):
    o = x0.shape[1]
    o2 = wf.shape[1]
    n = e // _BE
    blk = lambda j: pl.BlockSpec((_BE, o), lambda i, j=j: (j * n + i, 0))
    return pl.pallas_call(
        functools.partial(_conv1_body, float(e), n),
        grid=(n,),
        in_specs=[
            pl.BlockSpec((_BE, o), lambda i: (i, 0)),
            blk(0), blk(1), blk(2), blk(3),
            pl.BlockSpec((8, o), lambda i: (0, 0)),
            pl.BlockSpec((1, o), lambda i: (0, 0)),
            pl.BlockSpec((1, o), lambda i: (0, 0)),
            pl.BlockSpec((5 * o, o2), lambda i: (0, 0)),
            pl.BlockSpec((1, o2), lambda i: (0, 0)),
        ],
        out_specs=[
            pl.BlockSpec((_BE, o2), lambda i: (i, 0)),
            pl.BlockSpec((_BE, 1), lambda i: (i, 0)),
            pl.BlockSpec((8, o2), lambda i: (0, 0)),
        ],
        out_shape=[
            jax.ShapeDtypeStruct((e, o2), jnp.float32),
            jax.ShapeDtypeStruct((e, 1), jnp.float32),
            jax.ShapeDtypeStruct((8, o2), jnp.float32),
        ],
        compiler_params=pltpu.CompilerParams(
            dimension_semantics=("arbitrary",)),
    )(x0, g4, g4, g4, g4, mom, gam, bet, wf, bias)


def _head_body(nd_real, drop_ref, cs_ref, w1_ref, b1_ref, w2_ref, b2_ref,
               w3_ref, b3_ref, out_ref):
    nd_pad = drop_ref.shape[0]
    rows = lax.broadcasted_iota(jnp.int32, (nd_pad, 1), 0)
    mask = (rows < nd_real).astype(jnp.float32)
    dsum = jnp.sum(drop_ref[...] * mask, axis=0, keepdims=True)
    mean = (cs_ref[0:1, :] - dsum) * (1.0 / _E3)
    z = jnp.maximum(_dot3(mean, w1_ref[...]) + b1_ref[...], 0.0)
    z = jnp.maximum(_dot3(z, w2_ref[...]) + b2_ref[...], 0.0)
    z = _dot3(z, w3_ref[...]) + b3_ref[...]
    out_ref[...] = z


def _head(drop_rows, nd_real, cs, w1t, b1, w2t, b2, w3t, b3):
    nd = drop_rows.shape[0]
    return pl.pallas_call(
        functools.partial(_head_body, nd_real),
        grid=(1,),
        in_specs=[
            pl.BlockSpec((nd, 128), lambda i: (0, 0)),
            pl.BlockSpec((8, 128), lambda i: (0, 0)),
            pl.BlockSpec((128, 128), lambda i: (0, 0)),
            pl.BlockSpec((1, 128), lambda i: (0, 0)),
            pl.BlockSpec((128, 64), lambda i: (0, 0)),
            pl.BlockSpec((1, 64), lambda i: (0, 0)),
            pl.BlockSpec((64, 32), lambda i: (0, 0)),
            pl.BlockSpec((1, 32), lambda i: (0, 0)),
        ],
        out_specs=pl.BlockSpec((1, 32), lambda i: (0, 0)),
        out_shape=jax.ShapeDtypeStruct((1, 32), jnp.float32),
    )(drop_rows, cs, w1t, b1, w2t, b2, w3t, b3)


def _fold_w(w):
    o, c, _ = w.shape
    return jnp.transpose(w, (2, 1, 0)).reshape(5 * c, o)


def _res_block(xe, nbr, wf0, b0, wf1, b1, gam, bet, e):
    idx = jnp.transpose(nbr[0]).reshape(-1).astype(jnp.int32)
    c = xe.shape[1]
    g0 = _gather_rows(xe, idx)
    out0, mom = _conv0(xe, g0, wf0, b0, e)
    g1 = _gather_rows(out0, idx)
    return _conv1(out0, g1, mom, gam, bet, wf1, b1, e)


def kernel(x, nbr1, nbr2, nbr3, W1_0, b1_0, W1_1, b1_1, g1, be1,
           W2_0, b2_0, W2_1, b2_1, g2, be2, W3_0, b3_0, W3_1, b3_1, g3, be3,
           ffw1, ffb1, ffw2, ffb2, ffw3, ffb3):
    f32 = jnp.float32
    x0 = jnp.pad(x[0].T.astype(f32), ((0, 0), (0, 13)))
    w10 = jnp.pad(W1_0, ((0, 0), (0, 13), (0, 0)))
    wf10 = _fold_w(w10)
    row = lambda v: v.reshape(1, -1).astype(f32)

    out1, nrm1, _ = _res_block(x0, nbr1, wf10, row(b1_0), _fold_w(W1_1),
                               row(b1_1), row(g1), row(be1), _E0)
    _, i1 = lax.top_k(nrm1[:, 0], _E1)
    x2 = _gather_rows(out1, i1.astype(jnp.int32))

    out2, nrm2, _ = _res_block(x2, nbr2, _fold_w(W2_0), row(b2_0),
                               _fold_w(W2_1), row(b2_1), row(g2), row(be2), _E1)
    _, i2 = lax.top_k(nrm2[:, 0], _E2)
    x3 = _gather_rows(out2, i2.astype(jnp.int32))

    out3, nrm3, cs3 = _res_block(x3, nbr3, _fold_w(W3_0), row(b3_0),
                                 _fold_w(W3_1), row(b3_1), row(g3), row(be3),
                                 _E2)
    _, i3 = lax.top_k(nrm3[:, 0], _E2)
    dropped = i3[_E3:].astype(jnp.int32)
    drop_rows = _gather_rows(out3, dropped)

    return _head(drop_rows, _E2 - _E3, cs3, ffw1.T.astype(f32), row(ffb1),
                 ffw2.T.astype(f32), row(ffb2), ffw3.T.astype(f32), row(ffb3))


# speedup vs baseline: 13.6906x; 1.0404x over previous
import functools

import jax
import jax.numpy as jnp
from jax import lax
from jax.experimental import pallas as pl
from jax.experimental.pallas import tpu as pltpu
from jax.experimental.pallas import tpu_sc as plsc

_E0, _E1, _E2, _E3 = 160000, 144000, 129600, 116640
_BE = 1600
_NW = 32
_MAX_CHUNK_WORDS = 57344


def _pick_chunk(bpw, d):
    best = 0
    for ch in range(8, bpw + 1, 8):
        if bpw % ch == 0 and ch * d <= _MAX_CHUNK_WORDS:
            best = ch
    return best


def _plan_pad(b, d):
    best = (0, 0)
    bp = ((b + 255) // 256) * 256
    for _ in range(64):
        ch = _pick_chunk(bp // _NW, d)
        if ch >= 256:
            return bp, ch
        if ch > best[1]:
            best = (bp, ch)
        bp += 256
    return best


def _dot3(a, b):
    ah = a.astype(jnp.bfloat16)
    al = (a - ah.astype(jnp.float32)).astype(jnp.bfloat16)
    bh = b.astype(jnp.bfloat16)
    bl = (b - bh.astype(jnp.float32)).astype(jnp.bfloat16)
    kw = dict(preferred_element_type=jnp.float32)
    return (jnp.dot(ah, bl, **kw) + jnp.dot(al, bh, **kw)) + jnp.dot(ah, bh, **kw)


@functools.lru_cache(maxsize=None)
def _sc_gather_call(v, d, b, tc_tiling):
    assert b % (8 * _NW) == 0
    bpw = b // _NW
    ch = _pick_chunk(bpw, d)
    assert ch > 0, (v, d, b)
    nchunks = bpw // ch
    mesh = plsc.VectorSubcoreMesh(core_axis_name="c", subcore_axis_name="s")

    @functools.partial(
        pl.kernel,
        mesh=mesh,
        out_type=jax.ShapeDtypeStruct((b, d), jnp.float32),
        scratch_types=[
            pltpu.VMEM((2 * ch,), jnp.int32),
            pltpu.VMEM((2, ch, d), jnp.float32),
            pltpu.SemaphoreType.DMA,
            pltpu.SemaphoreType.DMA,
        ],
        compiler_params=pltpu.CompilerParams(use_tc_tiling_on_sc=tc_tiling),
    )
    def k(table_hbm, idx_hbm, out_hbm, idx_v, rows_v, gsem, wsem):
        wid = lax.axis_index("s") * 2 + lax.axis_index("c")
        base = wid * bpw
        pltpu.sync_copy(idx_hbm.at[pl.ds(base, ch)], idx_v.at[pl.ds(0, ch)])
        pltpu.async_copy(table_hbm.at[idx_v.at[pl.ds(0, ch)]], rows_v.at[0], gsem)

        def body(i, carry):
            cur = lax.rem(i, 2)
            nxt = lax.rem(i + 1, 2)
            off = base + i * ch
            pltpu.make_async_copy(
                table_hbm.at[idx_v.at[pl.ds(cur * ch, ch)]], rows_v.at[cur],
                gsem).wait()

            @pl.when(i >= 1)
            def _():
                pltpu.make_async_copy(
                    rows_v.at[nxt], out_hbm.at[pl.ds(off - ch, ch)], wsem).wait()

            @pl.when(i + 1 < nchunks)
            def _():
                pltpu.sync_copy(idx_hbm.at[pl.ds(off + ch, ch)],
                                idx_v.at[pl.ds(nxt * ch, ch)])
                pltpu.async_copy(table_hbm.at[idx_v.at[pl.ds(nxt * ch, ch)]],
                                 rows_v.at[nxt], gsem)

            pltpu.async_copy(rows_v.at[cur], out_hbm.at[pl.ds(off, ch)], wsem)
            return carry

        lax.fori_loop(0, nchunks, body, 0)
        last = nchunks - 1
        pltpu.make_async_copy(
            rows_v.at[last % 2],
            out_hbm.at[pl.ds(base + last * ch, ch)], wsem).wait()

    return k


def _gather_rows(table, idx):
    v, d = table.shape
    b = idx.shape[0]
    if b % 256 == 0 and _pick_chunk(b // _NW, d) > 0:
        bp = b
    else:
        bp, _ = _plan_pad(b, d)
    if bp != b:
        pad = jnp.arange(bp - b, dtype=jnp.int32) % v
        idx = jnp.concatenate([idx, pad])
    return _sc_gather_call(v, d, bp, d % 128 == 0)(table, idx)


def _conv0_body(nsteps, x_ref, a_ref, b_ref, c_ref, d_ref, wf_ref, bias_ref,
                out_ref, mom_ref):
    i = pl.program_id(0)
    x = x_ref[...]
    a, bb, cc, dd = a_ref[...], b_ref[...], c_ref[...], d_ref[...]
    feats = jnp.concatenate(
        [x, jnp.abs(a - cc), a + cc, jnp.abs(bb - dd), bb + dd], axis=1)
    out = _dot3(feats, wf_ref[...]) + bias_ref[...]
    out_ref[...] = out
    r = jnp.maximum(out, 0.0)

    @pl.when(i == 0)
    def _():
        mom_ref[...] = jnp.zeros_like(mom_ref)

    mom_ref[0:1, :] += jnp.sum(r, axis=0, keepdims=True)
    mom_ref[1:2, :] += jnp.sum(r * r, axis=0, keepdims=True)


def _conv1_body(e_edges, nsteps, x0_ref, a_ref, b_ref, c_ref, d_ref, mom_ref,
                g_ref, be_ref, wf_ref, bias_ref, out_ref, nrm_ref, cs_ref):
    i = pl.program_id(0)
    inv_e = 1.0 / e_edges
    m = mom_ref[0:1, :] * inv_e
    q = mom_ref[1:2, :] * inv_e
    var = q - m * m
    s = g_ref[...] * lax.rsqrt(var + 1e-5)
    t = be_ref[...] - m * s

    x0 = x0_ref[...]
    h = jnp.maximum(x0, 0.0) * s + t
    ha = jnp.maximum(a_ref[...], 0.0) * s + t
    hb = jnp.maximum(b_ref[...], 0.0) * s + t
    hc = jnp.maximum(c_ref[...], 0.0) * s + t
    hd = jnp.maximum(d_ref[...], 0.0) * s + t
    feats = jnp.concatenate(
        [h, jnp.abs(ha - hc), ha + hc, jnp.abs(hb - hd), hb + hd], axis=1)
    out = _dot3(feats, wf_ref[...]) + bias_ref[...] + x0
    out = jnp.maximum(out, 0.0)
    out_ref[...] = out
    nrm_ref[...] = jnp.sqrt(jnp.sum(out * out, axis=1))[:, None]

    @pl.when(i == 0)
    def _():
        cs_ref[...] = jnp.zeros_like(cs_ref)

    cs_ref[0:1, :] += jnp.sum(out, axis=0, keepdims=True)


def _conv0(xe, g4, wf, bias, e, be):
    c = xe.shape[1]
    o = wf.shape[1]
    n = e // be
    blk = lambda j: pl.BlockSpec((be, c), lambda i, j=j: (j * n + i, 0))
    return pl.pallas_call(
        functools.partial(_conv0_body, n),
        grid=(n,),
        in_specs=[
            pl.BlockSpec((be, c), lambda i: (i, 0)),
            blk(0), blk(1), blk(2), blk(3),
            pl.BlockSpec((5 * c, o), lambda i: (0, 0)),
            pl.BlockSpec((1, o), lambda i: (0, 0)),
        ],
        out_specs=[
            pl.BlockSpec((be, o), lambda i: (i, 0)),
            pl.BlockSpec((8, o), lambda i: (0, 0)),
        ],
        out_shape=[
            jax.ShapeDtypeStruct((e, o), jnp.float32),
            jax.ShapeDtypeStruct((8, o), jnp.float32),
        ],
        compiler_params=pltpu.CompilerParams(
            dimension_semantics=("arbitrary",)),
    )(xe, g4, g4, g4, g4, wf, bias)


def _conv1(x0, g4, mom, gam, bet, wf, bias, e, be):
    o = x0.shape[1]
    o2 = wf.shape[1]
    n = e // be
    blk = lambda j: pl.BlockSpec((be, o), lambda i, j=j: (j * n + i, 0))
    return pl.pallas_call(
        functools.partial(_conv1_body, float(e), n),
        grid=(n,),
        in_specs=[
            pl.BlockSpec((be, o), lambda i: (i, 0)),
            blk(0), blk(1), blk(2), blk(3),
            pl.BlockSpec((8, o), lambda i: (0, 0)),
            pl.BlockSpec((1, o), lambda i: (0, 0)),
            pl.BlockSpec((1, o), lambda i: (0, 0)),
            pl.BlockSpec((5 * o, o2), lambda i: (0, 0)),
            pl.BlockSpec((1, o2), lambda i: (0, 0)),
        ],
        out_specs=[
            pl.BlockSpec((be, o2), lambda i: (i, 0)),
            pl.BlockSpec((be, 1), lambda i: (i, 0)),
            pl.BlockSpec((8, o2), lambda i: (0, 0)),
        ],
        out_shape=[
            jax.ShapeDtypeStruct((e, o2), jnp.float32),
            jax.ShapeDtypeStruct((e, 1), jnp.float32),
            jax.ShapeDtypeStruct((8, o2), jnp.float32),
        ],
        compiler_params=pltpu.CompilerParams(
            dimension_semantics=("arbitrary",)),
    )(x0, g4, g4, g4, g4, mom, gam, bet, wf, bias)


def _head_body(nd_real, drop_ref, cs_ref, w1_ref, b1_ref, w2_ref, b2_ref,
               w3_ref, b3_ref, out_ref):
    nd_pad = drop_ref.shape[0]
    rows = lax.broadcasted_iota(jnp.int32, (nd_pad, 1), 0)
    mask = (rows < nd_real).astype(jnp.float32)
    dsum = jnp.sum(drop_ref[...] * mask, axis=0, keepdims=True)
    mean = (cs_ref[0:1, :] - dsum) * (1.0 / _E3)
    z = jnp.maximum(_dot3(mean, w1_ref[...]) + b1_ref[...], 0.0)
    z = jnp.maximum(_dot3(z, w2_ref[...]) + b2_ref[...], 0.0)
    z = _dot3(z, w3_ref[...]) + b3_ref[...]
    out_ref[...] = z


def _head(drop_rows, nd_real, cs, w1t, b1, w2t, b2, w3t, b3):
    nd = drop_rows.shape[0]
    return pl.pallas_call(
        functools.partial(_head_body, nd_real),
        grid=(1,),
        in_specs=[
            pl.BlockSpec((nd, 128), lambda i: (0, 0)),
            pl.BlockSpec((8, 128), lambda i: (0, 0)),
            pl.BlockSpec((128, 128), lambda i: (0, 0)),
            pl.BlockSpec((1, 128), lambda i: (0, 0)),
            pl.BlockSpec((128, 64), lambda i: (0, 0)),
            pl.BlockSpec((1, 64), lambda i: (0, 0)),
            pl.BlockSpec((64, 32), lambda i: (0, 0)),
            pl.BlockSpec((1, 32), lambda i: (0, 0)),
        ],
        out_specs=pl.BlockSpec((1, 32), lambda i: (0, 0)),
        out_shape=jax.ShapeDtypeStruct((1, 32), jnp.float32),
    )(drop_rows, cs, w1t, b1, w2t, b2, w3t, b3)


def _fold_w(w):
    o, c, _ = w.shape
    return jnp.transpose(w, (2, 1, 0)).reshape(5 * c, o)


def _res_block(xe, nbr, wf0, b0, wf1, b1, gam, bet, e):
    idx = jnp.transpose(nbr[0]).reshape(-1).astype(jnp.int32)
    be = 4000 if e % 4000 == 0 else _BE
    g0 = _gather_rows(xe, idx)
    out0, mom = _conv0(xe, g0, wf0, b0, e, be)
    g1 = _gather_rows(out0, idx)
    return _conv1(out0, g1, mom, gam, bet, wf1, b1, e, be)


def kernel(x, nbr1, nbr2, nbr3, W1_0, b1_0, W1_1, b1_1, g1, be1,
           W2_0, b2_0, W2_1, b2_1, g2, be2, W3_0, b3_0, W3_1, b3_1, g3, be3,
           ffw1, ffb1, ffw2, ffb2, ffw3, ffb3):
    f32 = jnp.float32
    x0 = jnp.pad(x[0].T.astype(f32), ((0, 0), (0, 13)))
    w10 = jnp.pad(W1_0, ((0, 0), (0, 13), (0, 0)))
    wf10 = _fold_w(w10)
    row = lambda v: v.reshape(1, -1).astype(f32)

    out1, nrm1, _ = _res_block(x0, nbr1, wf10, row(b1_0), _fold_w(W1_1),
                               row(b1_1), row(g1), row(be1), _E0)
    _, i1 = lax.top_k(nrm1[:, 0], _E1)
    x2 = _gather_rows(out1, i1.astype(jnp.int32))

    out2, nrm2, _ = _res_block(x2, nbr2, _fold_w(W2_0), row(b2_0),
                               _fold_w(W2_1), row(b2_1), row(g2), row(be2), _E1)
    _, i2 = lax.top_k(nrm2[:, 0], _E2)
    x3 = _gather_rows(out2, i2.astype(jnp.int32))

    out3, nrm3, cs3 = _res_block(x3, nbr3, _fold_w(W3_0), row(b3_0),
                                 _fold_w(W3_1), row(b3_1), row(g3), row(be3),
                                 _E2)
    _, i3 = lax.top_k(nrm3[:, 0], _E2)
    dropped = i3[_E3:].astype(jnp.int32)
    drop_rows = _gather_rows(out3, dropped)

    return _head(drop_rows, _E2 - _E3, cs3, ffw1.T.astype(f32), row(ffb1),
                 ffw2.T.astype(f32), row(ffb2), ffw3.T.astype(f32), row(ffb3))


# speedup vs baseline: 13.7952x; 1.0076x over previous
import functools

import jax
import jax.numpy as jnp
from jax import lax
from jax.experimental import pallas as pl
from jax.experimental.pallas import tpu as pltpu
from jax.experimental.pallas import tpu_sc as plsc

_E0, _E1, _E2, _E3 = 160000, 144000, 129600, 116640
_BE = 1600
_NW = 32
_MAX_CHUNK_WORDS = 57344


def _pick_chunk(bpw, d):
    best = 0
    for ch in range(8, bpw + 1, 8):
        if bpw % ch == 0 and ch * d <= _MAX_CHUNK_WORDS:
            best = ch
    return best


def _plan_pad(b, d):
    best = (0, 0)
    bp = ((b + 255) // 256) * 256
    for _ in range(64):
        ch = _pick_chunk(bp // _NW, d)
        if ch >= 256:
            return bp, ch
        if ch > best[1]:
            best = (bp, ch)
        bp += 256
    return best


def _dot3(a, b):
    ah = a.astype(jnp.bfloat16)
    al = (a - ah.astype(jnp.float32)).astype(jnp.bfloat16)
    bh = b.astype(jnp.bfloat16)
    bl = (b - bh.astype(jnp.float32)).astype(jnp.bfloat16)
    kw = dict(preferred_element_type=jnp.float32)
    return (jnp.dot(ah, bl, **kw) + jnp.dot(al, bh, **kw)) + jnp.dot(ah, bh, **kw)


@functools.lru_cache(maxsize=None)
def _sc_gather_call(v, d, b, tc_tiling):
    assert b % (8 * _NW) == 0
    bpw = b // _NW
    ch = _pick_chunk(bpw, d)
    assert ch > 0, (v, d, b)
    nchunks = bpw // ch
    mesh = plsc.VectorSubcoreMesh(core_axis_name="c", subcore_axis_name="s")

    @functools.partial(
        pl.kernel,
        mesh=mesh,
        out_type=jax.ShapeDtypeStruct((b, d), jnp.float32),
        scratch_types=[
            pltpu.VMEM((2 * ch,), jnp.int32),
            pltpu.VMEM((2, ch, d), jnp.float32),
            pltpu.SemaphoreType.DMA,
            pltpu.SemaphoreType.DMA,
        ],
        compiler_params=pltpu.CompilerParams(use_tc_tiling_on_sc=tc_tiling),
    )
    def k(table_hbm, idx_hbm, out_hbm, idx_v, rows_v, gsem, wsem):
        wid = lax.axis_index("s") * 2 + lax.axis_index("c")
        base = wid * bpw
        pltpu.sync_copy(idx_hbm.at[pl.ds(base, ch)], idx_v.at[pl.ds(0, ch)])
        pltpu.async_copy(table_hbm.at[idx_v.at[pl.ds(0, ch)]], rows_v.at[0], gsem)

        def body(i, carry):
            cur = lax.rem(i, 2)
            nxt = lax.rem(i + 1, 2)
            off = base + i * ch
            pltpu.make_async_copy(
                table_hbm.at[idx_v.at[pl.ds(cur * ch, ch)]], rows_v.at[cur],
                gsem).wait()

            @pl.when(i >= 1)
            def _():
                pltpu.make_async_copy(
                    rows_v.at[nxt], out_hbm.at[pl.ds(off - ch, ch)], wsem).wait()

            @pl.when(i + 1 < nchunks)
            def _():
                pltpu.sync_copy(idx_hbm.at[pl.ds(off + ch, ch)],
                                idx_v.at[pl.ds(nxt * ch, ch)])
                pltpu.async_copy(table_hbm.at[idx_v.at[pl.ds(nxt * ch, ch)]],
                                 rows_v.at[nxt], gsem)

            pltpu.async_copy(rows_v.at[cur], out_hbm.at[pl.ds(off, ch)], wsem)
            return carry

        lax.fori_loop(0, nchunks, body, 0)
        last = nchunks - 1
        pltpu.make_async_copy(
            rows_v.at[last % 2],
            out_hbm.at[pl.ds(base + last * ch, ch)], wsem).wait()

    return k


def _gather_rows(table, idx):
    v, d = table.shape
    b = idx.shape[0]
    if b % 256 == 0 and _pick_chunk(b // _NW, d) > 0:
        bp = b
    else:
        bp, _ = _plan_pad(b, d)
    if bp != b:
        pad = jnp.arange(bp - b, dtype=jnp.int32) % v
        idx = jnp.concatenate([idx, pad])
    return _sc_gather_call(v, d, bp, d % 128 == 0)(table, idx)


def _conv0_body(nsteps, x_ref, a_ref, b_ref, c_ref, d_ref, wf_ref, bias_ref,
                out_ref, mom_ref):
    i = pl.program_id(0)
    x = x_ref[...]
    a, bb, cc, dd = a_ref[...], b_ref[...], c_ref[...], d_ref[...]
    feats = jnp.concatenate(
        [x, jnp.abs(a - cc), a + cc, jnp.abs(bb - dd), bb + dd], axis=1)
    out = _dot3(feats, wf_ref[...]) + bias_ref[...]
    out_ref[...] = out
    r = jnp.maximum(out, 0.0)

    @pl.when(i == 0)
    def _():
        mom_ref[...] = jnp.zeros_like(mom_ref)

    mom_ref[0:1, :] += jnp.sum(r, axis=0, keepdims=True)
    mom_ref[1:2, :] += jnp.sum(r * r, axis=0, keepdims=True)


def _conv1_body(e_edges, nsteps, x0_ref, a_ref, b_ref, c_ref, d_ref, mom_ref,
                g_ref, be_ref, wf_ref, bias_ref, out_ref, nrm_ref, cs_ref):
    i = pl.program_id(0)
    inv_e = 1.0 / e_edges
    m = mom_ref[0:1, :] * inv_e
    q = mom_ref[1:2, :] * inv_e
    var = q - m * m
    s = g_ref[...] * lax.rsqrt(var + 1e-5)
    t = be_ref[...] - m * s

    x0 = x0_ref[...]
    h = jnp.maximum(x0, 0.0) * s + t
    ha = jnp.maximum(a_ref[...], 0.0) * s + t
    hb = jnp.maximum(b_ref[...], 0.0) * s + t
    hc = jnp.maximum(c_ref[...], 0.0) * s + t
    hd = jnp.maximum(d_ref[...], 0.0) * s + t
    feats = jnp.concatenate(
        [h, jnp.abs(ha - hc), ha + hc, jnp.abs(hb - hd), hb + hd], axis=1)
    out = _dot3(feats, wf_ref[...]) + bias_ref[...] + x0
    out = jnp.maximum(out, 0.0)
    out_ref[...] = out
    nrm_ref[...] = jnp.sqrt(jnp.sum(out * out, axis=1))[:, None]

    @pl.when(i == 0)
    def _():
        cs_ref[...] = jnp.zeros_like(cs_ref)

    cs_ref[0:1, :] += jnp.sum(out, axis=0, keepdims=True)


def _conv0(xe, g4, wf, bias, e, be):
    c = xe.shape[1]
    o = wf.shape[1]
    n = e // be
    blk = lambda j: pl.BlockSpec((be, c), lambda i, j=j: (j * n + i, 0))
    return pl.pallas_call(
        functools.partial(_conv0_body, n),
        grid=(n,),
        in_specs=[
            pl.BlockSpec((be, c), lambda i: (i, 0)),
            blk(0), blk(1), blk(2), blk(3),
            pl.BlockSpec((5 * c, o), lambda i: (0, 0)),
            pl.BlockSpec((1, o), lambda i: (0, 0)),
        ],
        out_specs=[
            pl.BlockSpec((be, o), lambda i: (i, 0)),
            pl.BlockSpec((8, o), lambda i: (0, 0)),
        ],
        out_shape=[
            jax.ShapeDtypeStruct((e, o), jnp.float32),
            jax.ShapeDtypeStruct((8, o), jnp.float32),
        ],
        compiler_params=pltpu.CompilerParams(
            dimension_semantics=("arbitrary",)),
    )(xe, g4, g4, g4, g4, wf, bias)


def _conv1(x0, g4, mom, gam, bet, wf, bias, e, be):
    o = x0.shape[1]
    o2 = wf.shape[1]
    n = e // be
    blk = lambda j: pl.BlockSpec((be, o), lambda i, j=j: (j * n + i, 0))
    return pl.pallas_call(
        functools.partial(_conv1_body, float(e), n),
        grid=(n,),
        in_specs=[
            pl.BlockSpec((be, o), lambda i: (i, 0)),
            blk(0), blk(1), blk(2), blk(3),
            pl.BlockSpec((8, o), lambda i: (0, 0)),
            pl.BlockSpec((1, o), lambda i: (0, 0)),
            pl.BlockSpec((1, o), lambda i: (0, 0)),
            pl.BlockSpec((5 * o, o2), lambda i: (0, 0)),
            pl.BlockSpec((1, o2), lambda i: (0, 0)),
        ],
        out_specs=[
            pl.BlockSpec((be, o2), lambda i: (i, 0)),
            pl.BlockSpec((be, 1), lambda i: (i, 0)),
            pl.BlockSpec((8, o2), lambda i: (0, 0)),
        ],
        out_shape=[
            jax.ShapeDtypeStruct((e, o2), jnp.float32),
            jax.ShapeDtypeStruct((e, 1), jnp.float32),
            jax.ShapeDtypeStruct((8, o2), jnp.float32),
        ],
        compiler_params=pltpu.CompilerParams(
            dimension_semantics=("arbitrary",)),
    )(x0, g4, g4, g4, g4, mom, gam, bet, wf, bias)


def _head_body(nd_real, drop_ref, cs_ref, w1_ref, b1_ref, w2_ref, b2_ref,
               w3_ref, b3_ref, out_ref):
    nd_pad = drop_ref.shape[0]
    rows = lax.broadcasted_iota(jnp.int32, (nd_pad, 1), 0)
    mask = (rows < nd_real).astype(jnp.float32)
    dsum = jnp.sum(drop_ref[...] * mask, axis=0, keepdims=True)
    mean = (cs_ref[0:1, :] - dsum) * (1.0 / _E3)
    z = jnp.maximum(_dot3(mean, w1_ref[...]) + b1_ref[...], 0.0)
    z = jnp.maximum(_dot3(z, w2_ref[...]) + b2_ref[...], 0.0)
    z = _dot3(z, w3_ref[...]) + b3_ref[...]
    out_ref[...] = z


def _head(drop_rows, nd_real, cs, w1t, b1, w2t, b2, w3t, b3):
    nd = drop_rows.shape[0]
    return pl.pallas_call(
        functools.partial(_head_body, nd_real),
        grid=(1,),
        in_specs=[
            pl.BlockSpec((nd, 128), lambda i: (0, 0)),
            pl.BlockSpec((8, 128), lambda i: (0, 0)),
            pl.BlockSpec((128, 128), lambda i: (0, 0)),
            pl.BlockSpec((1, 128), lambda i: (0, 0)),
            pl.BlockSpec((128, 64), lambda i: (0, 0)),
            pl.BlockSpec((1, 64), lambda i: (0, 0)),
            pl.BlockSpec((64, 32), lambda i: (0, 0)),
            pl.BlockSpec((1, 32), lambda i: (0, 0)),
        ],
        out_specs=pl.BlockSpec((1, 32), lambda i: (0, 0)),
        out_shape=jax.ShapeDtypeStruct((1, 32), jnp.float32),
    )(drop_rows, cs, w1t, b1, w2t, b2, w3t, b3)


def _fold_w(w):
    o, c, _ = w.shape
    return jnp.transpose(w, (2, 1, 0)).reshape(5 * c, o)


def _res_block(xe, nbr, wf0, b0, wf1, b1, gam, bet, e):
    idx = jnp.transpose(nbr[0]).reshape(-1).astype(jnp.int32)
    be = 4000 if e % 4000 == 0 else 2400
    g0 = _gather_rows(xe, idx)
    out0, mom = _conv0(xe, g0, wf0, b0, e, be)
    g1 = _gather_rows(out0, idx)
    return _conv1(out0, g1, mom, gam, bet, wf1, b1, e, be)


def kernel(x, nbr1, nbr2, nbr3, W1_0, b1_0, W1_1, b1_1, g1, be1,
           W2_0, b2_0, W2_1, b2_1, g2, be2, W3_0, b3_0, W3_1, b3_1, g3, be3,
           ffw1, ffb1, ffw2, ffb2, ffw3, ffb3):
    f32 = jnp.float32
    x0 = jnp.pad(x[0].T.astype(f32), ((0, 0), (0, 13)))
    w10 = jnp.pad(W1_0, ((0, 0), (0, 13), (0, 0)))
    wf10 = _fold_w(w10)
    row = lambda v: v.reshape(1, -1).astype(f32)

    out1, nrm1, _ = _res_block(x0, nbr1, wf10, row(b1_0), _fold_w(W1_1),
                               row(b1_1), row(g1), row(be1), _E0)
    _, i1 = lax.top_k(nrm1[:, 0], _E1)
    x2 = _gather_rows(out1, i1.astype(jnp.int32))

    out2, nrm2, _ = _res_block(x2, nbr2, _fold_w(W2_0), row(b2_0),
                               _fold_w(W2_1), row(b2_1), row(g2), row(be2), _E1)
    _, i2 = lax.top_k(nrm2[:, 0], _E2)
    x3 = _gather_rows(out2, i2.astype(jnp.int32))

    out3, nrm3, cs3 = _res_block(x3, nbr3, _fold_w(W3_0), row(b3_0),
                                 _fold_w(W3_1), row(b3_1), row(g3), row(be3),
                                 _E2)
    _, i3 = lax.top_k(nrm3[:, 0], _E2)
    dropped = i3[_E3:].astype(jnp.int32)
    drop_rows = _gather_rows(out3, dropped)

    return _head(drop_rows, _E2 - _E3, cs3, ffw1.T.astype(f32), row(ffb1),
                 ffw2.T.astype(f32), row(ffb2), ffw3.T.astype(f32), row(ffb3))
